# baseline (speedup 1.0000x reference)
import jax
import jax.numpy as jnp
from jax.experimental import pallas as pl
from jax.experimental.pallas import tpu as pltpu

T, D_MODEL, D_FF, E, TOP_K = 32, 768, 1536, 8, 2
HM = D_MODEL // 2
NF = 3
BF = D_FF // NF


def _moe_body(x_ref, ids_ref, tw_ref, w1_ref, w3_ref, w2a_ref, w2b_ref,
              out_ref, act_ref):
    e = pl.program_id(0)
    f = pl.program_id(1)

    @pl.when((e == 0) & (f == 0))
    def _init():
        out_ref[...] = jnp.zeros_like(out_ref)

    x = x_ref[...]
    h1 = jax.lax.dot_general(
        x, w1_ref[0, 0, 0], (((1,), (1,)), ((), ())),
        preferred_element_type=jnp.float32)
    h3 = jax.lax.dot_general(
        x, w3_ref[0, 0, 0], (((1,), (1,)), ((), ())),
        preferred_element_type=jnp.float32)
    act_ref[:, pl.ds(f * BF, BF)] = h1 * jax.nn.sigmoid(h1) * h3

    @pl.when(f == NF - 1)
    def _down():
        act = act_ref[...]
        oa = jax.lax.dot_general(
            act, w2a_ref[0, 0], (((1,), (1,)), ((), ())),
            preferred_element_type=jnp.float32)
        ob = jax.lax.dot_general(
            act, w2b_ref[0, 0], (((1,), (1,)), ((), ())),
            preferred_element_type=jnp.float32)
        gate = jnp.sum(
            jnp.where(ids_ref[...] == e, tw_ref[...], 0.0),
            axis=1, keepdims=True)
        out_ref[:, :HM] += gate * oa
        out_ref[:, HM:] += gate * ob


@jax.jit
def kernel(x, topk_ids, topk_weight, w13_weight, w2_weight):
    w13 = w13_weight.reshape(E, 2, NF, BF, D_MODEL)
    w2 = w2_weight.reshape(E, 2, HM, D_FF)
    return pl.pallas_call(
        _moe_body,
        grid=(E, NF),
        in_specs=[
            pl.BlockSpec((T, D_MODEL), lambda e, f: (0, 0)),
            pl.BlockSpec((T, TOP_K), lambda e, f: (0, 0)),
            pl.BlockSpec((T, TOP_K), lambda e, f: (0, 0)),
            pl.BlockSpec((1, 1, 1, BF, D_MODEL), lambda e, f: (e, 0, f, 0, 0)),
            pl.BlockSpec((1, 1, 1, BF, D_MODEL), lambda e, f: (e, 1, f, 0, 0)),
            pl.BlockSpec((1, 1, HM, D_FF), lambda e, f: (e, 0, 0, 0)),
            pl.BlockSpec((1, 1, HM, D_FF), lambda e, f: (e, 1, 0, 0)),
        ],
        out_specs=pl.BlockSpec((T, D_MODEL), lambda e, f: (0, 0)),
        out_shape=jax.ShapeDtypeStruct((T, D_MODEL), jnp.float32),
        scratch_shapes=[pltpu.VMEM((T, D_FF), jnp.float32)],
    )(x, topk_ids, topk_weight, w13, w13, w2, w2)


# speedup vs baseline: 1.0814x; 1.0814x over previous
import jax
import jax.numpy as jnp
from jax.experimental import pallas as pl
from jax.experimental.pallas import tpu as pltpu

T, D_MODEL, D_FF, E, TOP_K = 32, 768, 1536, 8, 2
HM = D_MODEL // 2
NBUF = 3


def _moe_body(x_ref, ids_ref, tw_ref, w13_hbm, w2_hbm, out_ref,
              w13_buf, w2_buf, sem13, sem2):
    def start(e):
        slot = e % NBUF
        pltpu.make_async_copy(
            w13_hbm.at[e], w13_buf.at[slot], sem13.at[slot]).start()
        pltpu.make_async_copy(
            w2_hbm.at[e], w2_buf.at[slot], sem2.at[slot]).start()

    def wait(e):
        slot = e % NBUF
        pltpu.make_async_copy(
            w13_hbm.at[e], w13_buf.at[slot], sem13.at[slot]).wait()
        pltpu.make_async_copy(
            w2_hbm.at[e], w2_buf.at[slot], sem2.at[slot]).wait()

    for e in range(NBUF):
        start(e)

    out_ref[...] = jnp.zeros_like(out_ref)
    x = x_ref[...]
    ids = ids_ref[...]
    tw = tw_ref[...]

    for e in range(E):
        slot = e % NBUF
        wait(e)
        h = jax.lax.dot_general(
            x, w13_buf[slot], (((1,), (1,)), ((), ())),
            preferred_element_type=jnp.float32)
        h1 = h[:, :D_FF]
        h3 = h[:, D_FF:]
        act = h1 * jax.nn.sigmoid(h1) * h3
        o = jax.lax.dot_general(
            act, w2_buf[slot], (((1,), (1,)), ((), ())),
            preferred_element_type=jnp.float32)
        gate = jnp.sum(
            jnp.where(ids == e, tw, 0.0), axis=1, keepdims=True)
        out_ref[...] += gate * o
        if e + NBUF < E:
            start(e + NBUF)


@jax.jit
def kernel(x, topk_ids, topk_weight, w13_weight, w2_weight):
    return pl.pallas_call(
        _moe_body,
        in_specs=[
            pl.BlockSpec(memory_space=pltpu.VMEM),
            pl.BlockSpec(memory_space=pltpu.VMEM),
            pl.BlockSpec(memory_space=pltpu.VMEM),
            pl.BlockSpec(memory_space=pl.ANY),
            pl.BlockSpec(memory_space=pl.ANY),
        ],
        out_specs=pl.BlockSpec(memory_space=pltpu.VMEM),
        out_shape=jax.ShapeDtypeStruct((T, D_MODEL), jnp.float32),
        scratch_shapes=[
            pltpu.VMEM((NBUF, 2 * D_FF, D_MODEL), jnp.float32),
            pltpu.VMEM((NBUF, D_MODEL, D_FF), jnp.float32),
            pltpu.SemaphoreType.DMA((NBUF,)),
            pltpu.SemaphoreType.DMA((NBUF,)),
        ],
    )(x, topk_ids, topk_weight, w13_weight, w2_weight)


# speedup vs baseline: 1.1143x; 1.0304x over previous
import jax
import jax.numpy as jnp
from jax.experimental import pallas as pl
from jax.experimental.pallas import tpu as pltpu

T, D_MODEL, D_FF, E, TOP_K = 32, 768, 1536, 8, 2
NBUF = 3


def _moe_body(x_ref, ids_ref, tw_ref, w13_hbm, w2_hbm, out_ref,
              w1_buf, w3_buf, w2_buf, sem1, sem3, sem2):
    def cps(e):
        slot = e % NBUF
        return (
            pltpu.make_async_copy(
                w13_hbm.at[e, 0], w1_buf.at[slot], sem1.at[slot]),
            pltpu.make_async_copy(
                w13_hbm.at[e, 1], w3_buf.at[slot], sem3.at[slot]),
            pltpu.make_async_copy(
                w2_hbm.at[e], w2_buf.at[slot], sem2.at[slot]),
        )

    def start(e):
        for c in cps(e):
            c.start()

    for e in range(NBUF):
        start(e)

    out_ref[...] = jnp.zeros_like(out_ref)
    x = x_ref[...]
    ids = ids_ref[...]
    tw = tw_ref[...]

    for e in range(E):
        slot = e % NBUF
        c1, c3, c2 = cps(e)
        c1.wait()
        h1 = jax.lax.dot_general(
            x, w1_buf[slot], (((1,), (1,)), ((), ())),
            preferred_element_type=jnp.float32)
        c3.wait()
        h3 = jax.lax.dot_general(
            x, w3_buf[slot], (((1,), (1,)), ((), ())),
            preferred_element_type=jnp.float32)
        act = h1 * jax.nn.sigmoid(h1) * h3
        c2.wait()
        o = jax.lax.dot_general(
            act, w2_buf[slot], (((1,), (1,)), ((), ())),
            preferred_element_type=jnp.float32)
        gate = jnp.sum(
            jnp.where(ids == e, tw, 0.0), axis=1, keepdims=True)
        out_ref[...] += gate * o
        if e + NBUF < E:
            start(e + NBUF)


@jax.jit
def kernel(x, topk_ids, topk_weight, w13_weight, w2_weight):
    w13 = w13_weight.reshape(E, 2, D_FF, D_MODEL)
    return pl.pallas_call(
        _moe_body,
        in_specs=[
            pl.BlockSpec(memory_space=pltpu.VMEM),
            pl.BlockSpec(memory_space=pltpu.VMEM),
            pl.BlockSpec(memory_space=pltpu.VMEM),
            pl.BlockSpec(memory_space=pl.ANY),
            pl.BlockSpec(memory_space=pl.ANY),
        ],
        out_specs=pl.BlockSpec(memory_space=pltpu.VMEM),
        out_shape=jax.ShapeDtypeStruct((T, D_MODEL), jnp.float32),
        scratch_shapes=[
            pltpu.VMEM((NBUF, D_FF, D_MODEL), jnp.float32),
            pltpu.VMEM((NBUF, D_FF, D_MODEL), jnp.float32),
            pltpu.VMEM((NBUF, D_MODEL, D_FF), jnp.float32),
            pltpu.SemaphoreType.DMA((NBUF,)),
            pltpu.SemaphoreType.DMA((NBUF,)),
            pltpu.SemaphoreType.DMA((NBUF,)),
        ],
    )(x, topk_ids, topk_weight, w13, w2_weight)
